# baseline (speedup 1.0000x reference)
import functools

import jax
import jax.numpy as jnp
from jax.experimental import pallas as pl
from jax.experimental.pallas import tpu as pltpu

_B = 64
_V = 32768
_K = 8
_ROWS = 256
_LANES = 1024
_JCOL = 128


def _tc_body(x_ref, u1_ref, u2_ref, z_ref, z2_ref, s1_ref, s2_ref):
    x = x_ref[0]
    x = jnp.where(jnp.isneginf(x), 0.0, x)
    m_row = jax.lax.broadcasted_iota(jnp.int32, (_LANES, _JCOL), 0)
    j_col = jax.lax.broadcasted_iota(jnp.int32, (_LANES, _JCOL), 1)
    sel = ((m_row >> 3) == j_col).astype(jnp.float32)
    l = jax.lax.dot_general(
        x, sel, (((1,), (0,)), ((), ())),
        preferred_element_type=jnp.float32,
        precision=jax.lax.Precision.HIGHEST,
    )
    z = jnp.where(l == 0.0, -1000.0, l)
    z_ref[0] = z
    z2_ref[0] = z

    lin = (jax.lax.broadcasted_iota(jnp.int32, (_ROWS, _JCOL), 0) * _JCOL
           + jax.lax.broadcasted_iota(jnp.int32, (_ROWS, _JCOL), 1))

    def samp(u):
        g = -jnp.log(-jnp.log(u))
        n = z + g
        mx = jnp.max(n)
        idx = jnp.min(jnp.where(n == mx, lin, jnp.int32(2**30)))
        return idx

    s1_ref[0, 0] = jnp.full((_JCOL,), samp(u1_ref[0]), dtype=jnp.int32)
    s2_ref[0, 0] = jnp.full((_JCOL,), samp(u2_ref[0]), dtype=jnp.int32)


@functools.partial(jax.jit)
def _run(logits, u1, u2):
    x3 = logits.reshape(_B, _ROWS, _LANES)
    u1r = u1.reshape(_B, _ROWS, _JCOL)
    u2r = u2.reshape(_B, _ROWS, _JCOL)
    z, z2, s1, s2 = pl.pallas_call(
        _tc_body,
        grid=(_B,),
        in_specs=[
            pl.BlockSpec((1, _ROWS, _LANES), lambda b: (b, 0, 0)),
            pl.BlockSpec((1, _ROWS, _JCOL), lambda b: (b, 0, 0)),
            pl.BlockSpec((1, _ROWS, _JCOL), lambda b: (b, 0, 0)),
        ],
        out_specs=[
            pl.BlockSpec((1, _ROWS, _JCOL), lambda b: (b, 0, 0)),
            pl.BlockSpec((1, _ROWS, _JCOL), lambda b: (b, 0, 0)),
            pl.BlockSpec((1, 1, _JCOL), lambda b: (b, 0, 0)),
            pl.BlockSpec((1, 1, _JCOL), lambda b: (b, 0, 0)),
        ],
        out_shape=[
            jax.ShapeDtypeStruct((_B, _ROWS, _JCOL), jnp.float32),
            jax.ShapeDtypeStruct((_B, _ROWS, _JCOL), jnp.float32),
            jax.ShapeDtypeStruct((_B, 1, _JCOL), jnp.int32),
            jax.ShapeDtypeStruct((_B, 1, _JCOL), jnp.int32),
        ],
        compiler_params=pltpu.CompilerParams(
            dimension_semantics=("parallel",),
        ),
    )(x3, u1r, u2r)
    return (z.reshape(_B, _V), z2.reshape(_B, _V), s1[:, 0, 0], s2[:, 0, 0])


def kernel(logits):
    k1, k2 = jax.random.split(jax.random.key(42))
    u1 = jax.random.uniform(k1, (_B, _V), jnp.float32, minval=1e-20, maxval=1.0)
    u2 = jax.random.uniform(k2, (_B, _V), jnp.float32, minval=1e-20, maxval=1.0)
    return _run(logits, u1, u2)


# speedup vs baseline: 1.8576x; 1.8576x over previous
import functools

import jax
import jax.numpy as jnp
from jax.experimental import pallas as pl
from jax.experimental.pallas import tpu as pltpu

_B = 64
_V = 32768
_K = 8
_BG = 8
_G = _B // _BG


def _tc_body(x_ref, u1_ref, u2_ref, z_ref, z2_ref, s1_ref, s2_ref):
    x = x_ref[...]
    x = jnp.where(jnp.isneginf(x), 0.0, x)
    l = jnp.sum(x, axis=1)
    z = jnp.where(l == 0.0, -1000.0, l)
    z_ref[...] = z
    z2_ref[...] = z

    lin = jax.lax.broadcasted_iota(jnp.int32, (_BG, _V), 1)

    def samp(u):
        g = -jnp.log(-jnp.log(u))
        n = z + g
        mx = jnp.max(n, axis=1, keepdims=True)
        idx = jnp.min(jnp.where(n == mx, lin, jnp.int32(2**30)),
                      axis=1, keepdims=True)
        return jnp.broadcast_to(idx, (_BG, 128)).astype(jnp.int32)

    s1_ref[0] = samp(u1_ref[...])
    s2_ref[0] = samp(u2_ref[...])


@functools.partial(jax.jit)
def _run(logits, u1, u2):
    xt = jnp.swapaxes(logits, 1, 2)
    z, z2, s1, s2 = pl.pallas_call(
        _tc_body,
        grid=(_G,),
        in_specs=[
            pl.BlockSpec((_BG, _K, _V), lambda g: (g, 0, 0)),
            pl.BlockSpec((_BG, _V), lambda g: (g, 0)),
            pl.BlockSpec((_BG, _V), lambda g: (g, 0)),
        ],
        out_specs=[
            pl.BlockSpec((_BG, _V), lambda g: (g, 0)),
            pl.BlockSpec((_BG, _V), lambda g: (g, 0)),
            pl.BlockSpec((1, _BG, 128), lambda g: (g, 0, 0)),
            pl.BlockSpec((1, _BG, 128), lambda g: (g, 0, 0)),
        ],
        out_shape=[
            jax.ShapeDtypeStruct((_B, _V), jnp.float32),
            jax.ShapeDtypeStruct((_B, _V), jnp.float32),
            jax.ShapeDtypeStruct((_G, _BG, 128), jnp.int32),
            jax.ShapeDtypeStruct((_G, _BG, 128), jnp.int32),
        ],
        compiler_params=pltpu.CompilerParams(
            dimension_semantics=("parallel",),
        ),
    )(xt, u1, u2)
    return (z, z2, s1[:, :, 0].reshape(_B), s2[:, :, 0].reshape(_B))


def kernel(logits):
    k1, k2 = jax.random.split(jax.random.key(42))
    u1 = jax.random.uniform(k1, (_B, _V), jnp.float32, minval=1e-20, maxval=1.0)
    u2 = jax.random.uniform(k2, (_B, _V), jnp.float32, minval=1e-20, maxval=1.0)
    return _run(logits, u1, u2)
